# baseline (device time: 13258 ns/iter reference)
import jax
import jax.numpy as jnp
from jax import lax
from jax.experimental import pallas as pl
from jax.experimental.pallas import tpu as pltpu

N_DEV = 8
B, SQ, D_MODEL, HQ, DH = 2, 256, 512, 4, 64
BLK = 64
BSQ = B * SQ
NCHUNK = HQ
NEDGE = 3
NONE = N_DEV
BF = jnp.bfloat16
F32 = jnp.float32


def kernel(x, Wq, K_ext, V_ext, Wo):
    hbm = pltpu.MemorySpace.HBM
    kt = pltpu.with_memory_space_constraint(
        jnp.transpose(K_ext, (0, 2, 3, 1)), hbm)
    vt = pltpu.with_memory_space_constraint(
        jnp.transpose(V_ext, (0, 2, 3, 1)), hbm)
    xc = pltpu.with_memory_space_constraint(x.reshape(BSQ, D_MODEL), hbm)
    wqc = pltpu.with_memory_space_constraint(Wq, hbm)
    woc = pltpu.with_memory_space_constraint(Wo, hbm)

    def body(x_ref, wq_ref, k_ref, v_ref, wo_ref, out_ref,
             ctx_ref, q_ref, xw_ref, wq_vref, kv_ref, wo_vref,
             send_sems, recv_sems, copy_sems):
        my_i = lax.axis_index("i")

        cp_wo = pltpu.make_async_copy(wo_ref, wo_vref, copy_sems.at[4])
        cp_wo.start()

        cps = [
            pltpu.make_async_copy(x_ref, xw_ref, copy_sems.at[0]),
            pltpu.make_async_copy(wq_ref, wq_vref, copy_sems.at[1]),
            pltpu.make_async_copy(k_ref, kv_ref.at[0], copy_sems.at[2]),
            pltpu.make_async_copy(v_ref, kv_ref.at[1], copy_sems.at[3]),
        ]

        @pl.when(my_i == 0)
        def _issue_input_copies():
            for cp in cps:
                cp.start()

        tgts = [
            jnp.where(my_i == 0, 4,
            jnp.where(my_i == 1, 2,
            jnp.where(my_i == 4, 5,
            jnp.where(my_i == 5, 6, NONE)))),
            jnp.where(my_i == 0, 1, jnp.where(my_i == 4, 7, NONE)),
            jnp.where(my_i == 0, 3, NONE),
        ]
        parent = jnp.where(my_i == 1, 0,
                 jnp.where(my_i == 2, 1,
                 jnp.where(my_i == 3, 0,
                 jnp.where(my_i == 4, 0,
                 jnp.where(my_i == 5, 4,
                 jnp.where(my_i == 6, 5,
                 jnp.where(my_i == 7, 4, NONE)))))))

        barrier = pltpu.get_barrier_semaphore()
        for e in range(NEDGE):
            @pl.when(tgts[e] < N_DEV)
            def _sig(t=tgts[e]):
                pl.semaphore_signal(
                    barrier, inc=1, device_id=(t,),
                    device_id_type=pl.DeviceIdType.MESH,
                )

        @pl.when(parent < N_DEV)
        def _sig_parent():
            pl.semaphore_signal(
                barrier, inc=1, device_id=(parent,),
                device_id_type=pl.DeviceIdType.MESH,
            )

        @pl.when((my_i == 0) | (my_i == 4))
        def _wait3():
            pl.semaphore_wait(barrier, 3)

        @pl.when((my_i == 1) | (my_i == 5))
        def _wait2():
            pl.semaphore_wait(barrier, 2)

        @pl.when((my_i == 2) | (my_i == 3) | (my_i == 6) | (my_i == 7))
        def _wait1():
            pl.semaphore_wait(barrier, 1)

        q_blk = lax.broadcasted_iota(jnp.int32, (SQ, SQ), 0) // BLK
        k_blk = lax.broadcasted_iota(jnp.int32, (SQ, SQ), 1) // BLK
        mask = k_blk <= q_blk
        cp_wo.wait()
        wo = wo_vref[...].astype(BF)

        for h in range(NCHUNK):
            @pl.when(my_i == 0)
            def _compute(h=h):
                if h == 0:
                    cps[0].wait()
                    cps[1].wait()
                    q_ref[...] = lax.dot_general(
                        xw_ref[...].astype(BF), wq_vref[...].astype(BF),
                        (((1,), (0,)), ((), ())),
                        preferred_element_type=F32,
                    ).astype(BF)
                    cps[2].wait()
                    cps[3].wait()
                for b in range(B):
                    q_h = q_ref[b * SQ:(b + 1) * SQ, h * DH:(h + 1) * DH]
                    k_h = kv_ref[0, b, h].astype(BF)
                    v_h = kv_ref[1, b, h].astype(BF)
                    s = lax.dot_general(
                        q_h, k_h, (((1,), (0,)), ((), ())),
                        preferred_element_type=F32,
                    ) * 0.125
                    w = jnp.exp(jnp.where(mask, s, -1e9))
                    r = 1.0 / jnp.sum(w, axis=-1, keepdims=True)
                    cv = lax.dot_general(
                        w.astype(BF), v_h, (((1,), (1,)), ((), ())),
                        preferred_element_type=F32,
                    ) * r
                    ctx_ref[h, b * SQ:(b + 1) * SQ, :] = cv.astype(BF)

            @pl.when(my_i > 0)
            def _recv(h=h):
                pltpu.make_async_remote_copy(
                    src_ref=ctx_ref.at[h], dst_ref=ctx_ref.at[h],
                    send_sem=send_sems.at[0, h], recv_sem=recv_sems.at[h],
                    device_id=(0,), device_id_type=pl.DeviceIdType.MESH,
                ).wait_recv()

            for e in range(NEDGE):
                @pl.when(tgts[e] < N_DEV)
                def _send(t=tgts[e], e=e, h=h):
                    pltpu.make_async_remote_copy(
                        src_ref=ctx_ref.at[h], dst_ref=ctx_ref.at[h],
                        send_sem=send_sems.at[e, h], recv_sem=recv_sems.at[h],
                        device_id=(t,), device_id_type=pl.DeviceIdType.MESH,
                    ).start()

            @pl.when(my_i > 0)
            def _acc(h=h):
                d = jnp.dot(
                    ctx_ref[h], wo[h * DH:(h + 1) * DH, :],
                    preferred_element_type=F32,
                )
                for b in range(B):
                    db = d[b * SQ:(b + 1) * SQ, :]
                    if h == 0:
                        out_ref[b] = db
                    else:
                        out_ref[b] = out_ref[b] + db

        @pl.when(my_i == 0)
        def _acc_dev0():
            for b in range(B):
                out_b = jnp.zeros((SQ, D_MODEL), F32)
                for h in range(HQ):
                    out_b = out_b + jnp.dot(
                        ctx_ref[h, b * SQ:(b + 1) * SQ, :],
                        wo[h * DH:(h + 1) * DH, :],
                        preferred_element_type=F32,
                    )
                out_ref[b] = out_b

        for h in range(NCHUNK):
            for e in range(NEDGE):
                @pl.when(tgts[e] < N_DEV)
                def _wait(t=tgts[e], e=e, h=h):
                    pltpu.make_async_remote_copy(
                        src_ref=ctx_ref.at[h], dst_ref=ctx_ref.at[h],
                        send_sem=send_sems.at[e, h], recv_sem=recv_sems.at[h],
                        device_id=(t,), device_id_type=pl.DeviceIdType.MESH,
                    ).wait_send()

    out_shape = jax.ShapeDtypeStruct((B, SQ, D_MODEL), F32)
    return pl.pallas_call(
        body,
        out_shape=out_shape,
        in_specs=[pl.BlockSpec(memory_space=pl.ANY)] * 5,
        out_specs=pl.BlockSpec(memory_space=pltpu.VMEM),
        scratch_shapes=[
            pltpu.VMEM((NCHUNK, BSQ, DH), BF),
            pltpu.VMEM((BSQ, HQ * DH), BF),
            pltpu.VMEM((BSQ, D_MODEL), F32),
            pltpu.VMEM((D_MODEL, HQ * DH), F32),
            pltpu.VMEM((2, B, HQ, DH, SQ), F32),
            pltpu.VMEM((HQ * DH, D_MODEL), F32),
            pltpu.SemaphoreType.DMA((NEDGE, NCHUNK)),
            pltpu.SemaphoreType.DMA((NCHUNK,)),
            pltpu.SemaphoreType.DMA((5,)),
        ],
        compiler_params=pltpu.CompilerParams(collective_id=0),
    )(xc, wqc, kt, vt, woc)


# device time: 11625 ns/iter; 1.1405x vs baseline; 1.1405x over previous
import jax
import jax.numpy as jnp
from jax import lax
from jax.experimental import pallas as pl
from jax.experimental.pallas import tpu as pltpu

N_DEV = 8
B, SQ, D_MODEL, HQ, DH = 2, 256, 512, 4, 64
BLK = 64
NCHUNK = B * HQ
NEDGE = 3
NONE = N_DEV
BF = jnp.bfloat16
F32 = jnp.float32


def kernel(x, Wq, K_ext, V_ext, Wo):
    hbm = pltpu.MemorySpace.HBM
    kt = pltpu.with_memory_space_constraint(
        jnp.transpose(K_ext, (0, 2, 3, 1)), hbm)
    vt = pltpu.with_memory_space_constraint(
        jnp.transpose(V_ext, (0, 2, 3, 1)), hbm)
    xc = pltpu.with_memory_space_constraint(x, hbm)
    wqc = pltpu.with_memory_space_constraint(Wq, hbm)
    woc = pltpu.with_memory_space_constraint(Wo, hbm)

    def body(x_ref, wq_ref, k_ref, v_ref, wo_ref, out_ref,
             ctx_ref, q_ref, xw_ref, wq_vref, kv_ref, wo_vref,
             send_sems, recv_sems, copy_sems):
        my_i = lax.axis_index("i")

        cp_wo = pltpu.make_async_copy(wo_ref, wo_vref, copy_sems.at[4])
        cp_wo.start()

        cps = [
            pltpu.make_async_copy(x_ref, xw_ref, copy_sems.at[0]),
            pltpu.make_async_copy(wq_ref, wq_vref, copy_sems.at[1]),
            pltpu.make_async_copy(k_ref, kv_ref.at[0], copy_sems.at[2]),
            pltpu.make_async_copy(v_ref, kv_ref.at[1], copy_sems.at[3]),
        ]

        @pl.when(my_i == 0)
        def _issue_input_copies():
            for cp in cps:
                cp.start()

        tgts = [
            jnp.where(my_i == 0, 4,
            jnp.where(my_i == 1, 2,
            jnp.where(my_i == 4, 5,
            jnp.where(my_i == 5, 6, NONE)))),
            jnp.where(my_i == 0, 1, jnp.where(my_i == 4, 7, NONE)),
            jnp.where(my_i == 0, 3, NONE),
        ]
        parent = jnp.where(my_i == 1, 0,
                 jnp.where(my_i == 2, 1,
                 jnp.where(my_i == 3, 0,
                 jnp.where(my_i == 4, 0,
                 jnp.where(my_i == 5, 4,
                 jnp.where(my_i == 6, 5,
                 jnp.where(my_i == 7, 4, NONE)))))))

        barrier = pltpu.get_barrier_semaphore()
        for e in range(NEDGE):
            @pl.when(tgts[e] < N_DEV)
            def _sig(t=tgts[e]):
                pl.semaphore_signal(
                    barrier, inc=1, device_id=(t,),
                    device_id_type=pl.DeviceIdType.MESH,
                )

        @pl.when(parent < N_DEV)
        def _sig_parent():
            pl.semaphore_signal(
                barrier, inc=1, device_id=(parent,),
                device_id_type=pl.DeviceIdType.MESH,
            )

        @pl.when(my_i == 4)
        def _wait3():
            pl.semaphore_wait(barrier, 3)

        @pl.when((my_i == 1) | (my_i == 5))
        def _wait2():
            pl.semaphore_wait(barrier, 2)

        @pl.when((my_i == 2) | (my_i == 3) | (my_i == 6) | (my_i == 7))
        def _wait1():
            pl.semaphore_wait(barrier, 1)

        q_blk = lax.broadcasted_iota(jnp.int32, (SQ, SQ), 0) // BLK
        k_blk = lax.broadcasted_iota(jnp.int32, (SQ, SQ), 1) // BLK
        mask = k_blk <= q_blk
        cp_wo.wait()
        wo = wo_vref[...].astype(BF)

        @pl.when(my_i == 0)
        def _compute_all():
            cps[0].wait()
            cps[1].wait()
            wq_bf = wq_vref[...].astype(BF)
            for b in range(B):
                q_ref[b] = lax.dot_general(
                    xw_ref[b].astype(BF), wq_bf,
                    (((1,), (0,)), ((), ())),
                    preferred_element_type=F32,
                ).astype(BF)
            cps[2].wait()
            cps[3].wait()
            for b in range(B):
                for h in range(HQ):
                    q_h = q_ref[b, :, h * DH:(h + 1) * DH]
                    k_h = kv_ref[0, b, h].astype(BF)
                    v_h = kv_ref[1, b, h].astype(BF)
                    s = lax.dot_general(
                        q_h, k_h, (((1,), (0,)), ((), ())),
                        preferred_element_type=F32,
                    ) * 0.125
                    w = jnp.exp(jnp.where(mask, s, -1e9))
                    r = 1.0 / jnp.sum(w, axis=-1, keepdims=True)
                    cv = lax.dot_general(
                        w.astype(BF), v_h, (((1,), (1,)), ((), ())),
                        preferred_element_type=F32,
                    ) * r
                    ctx_ref[b, h] = cv.astype(BF)

        @pl.when(my_i == 0)
        def _wait_barrier_dev0():
            pl.semaphore_wait(barrier, 3)

        for c in range(NCHUNK):
            b, h = divmod(c, HQ)

            @pl.when(my_i > 0)
            def _recv(c=c, b=b, h=h):
                pltpu.make_async_remote_copy(
                    src_ref=ctx_ref.at[b, h], dst_ref=ctx_ref.at[b, h],
                    send_sem=send_sems.at[0, c], recv_sem=recv_sems.at[c],
                    device_id=(0,), device_id_type=pl.DeviceIdType.MESH,
                ).wait_recv()

            for e in range(NEDGE):
                @pl.when(tgts[e] < N_DEV)
                def _send(t=tgts[e], e=e, c=c, b=b, h=h):
                    pltpu.make_async_remote_copy(
                        src_ref=ctx_ref.at[b, h], dst_ref=ctx_ref.at[b, h],
                        send_sem=send_sems.at[e, c], recv_sem=recv_sems.at[c],
                        device_id=(t,), device_id_type=pl.DeviceIdType.MESH,
                    ).start()

            @pl.when(my_i > 0)
            def _acc(b=b, h=h):
                d = jnp.dot(
                    ctx_ref[b, h], wo[h * DH:(h + 1) * DH, :],
                    preferred_element_type=F32,
                )
                if h == 0:
                    out_ref[b] = d
                else:
                    out_ref[b] = out_ref[b] + d

        @pl.when(my_i == 0)
        def _acc_dev0():
            for b in range(B):
                out_b = jnp.zeros((SQ, D_MODEL), F32)
                for h in range(HQ):
                    out_b = out_b + jnp.dot(
                        ctx_ref[b, h], wo[h * DH:(h + 1) * DH, :],
                        preferred_element_type=F32,
                    )
                out_ref[b] = out_b

        for c in range(NCHUNK):
            b, h = divmod(c, HQ)
            for e in range(NEDGE):
                @pl.when(tgts[e] < N_DEV)
                def _wait(t=tgts[e], e=e, c=c, b=b, h=h):
                    pltpu.make_async_remote_copy(
                        src_ref=ctx_ref.at[b, h], dst_ref=ctx_ref.at[b, h],
                        send_sem=send_sems.at[e, c], recv_sem=recv_sems.at[c],
                        device_id=(t,), device_id_type=pl.DeviceIdType.MESH,
                    ).wait_send()

    out_shape = jax.ShapeDtypeStruct((B, SQ, D_MODEL), F32)
    return pl.pallas_call(
        body,
        out_shape=out_shape,
        in_specs=[pl.BlockSpec(memory_space=pl.ANY)] * 5,
        out_specs=pl.BlockSpec(memory_space=pltpu.VMEM),
        scratch_shapes=[
            pltpu.VMEM((B, HQ, SQ, DH), BF),
            pltpu.VMEM((B, SQ, HQ * DH), BF),
            pltpu.VMEM((B, SQ, D_MODEL), F32),
            pltpu.VMEM((D_MODEL, HQ * DH), F32),
            pltpu.VMEM((2, B, HQ, DH, SQ), F32),
            pltpu.VMEM((HQ * DH, D_MODEL), F32),
            pltpu.SemaphoreType.DMA((NEDGE, NCHUNK)),
            pltpu.SemaphoreType.DMA((NCHUNK,)),
            pltpu.SemaphoreType.DMA((5,)),
        ],
        compiler_params=pltpu.CompilerParams(collective_id=0),
    )(xc, wqc, kt, vt, woc)
